# baseline (speedup 1.0000x reference)
import functools

import jax
import jax.numpy as jnp
from jax import lax
from jax.experimental import pallas as pl
from jax.experimental.pallas import tpu as pltpu
from jax.experimental.pallas import tpu_sc as plsc

LANES = 16
BT = 128


@functools.cache
def _make_gather(s: int, b: int, d: int):
    info = plsc.get_sparse_core_info()
    nw = info.num_cores * info.num_subcores
    assert b == BT * nw and d == 64 and s % 2 == 0

    mesh = plsc.VectorSubcoreMesh(core_axis_name="c", subcore_axis_name="s")

    @functools.partial(
        pl.kernel,
        mesh=mesh,
        out_type=jax.ShapeDtypeStruct((s, d, b), jnp.float32),
        scratch_types=[
            pltpu.VMEM((s, BT), jnp.int32),
            pltpu.VMEM((2, BT), jnp.int32),
            pltpu.VMEM((2, BT, 2 * d), jnp.float32),
            pltpu.VMEM((2, d, BT), jnp.float32),
        ]
        + [pltpu.SemaphoreType.DMA] * 4,
        compiler_params=pltpu.CompilerParams(needs_layout_passes=False),
    )
    def gather_kernel(table_hbm, idx_hbm, out_hbm, idx_v, ridx_v, rows_v, out_v, *sems):
        gsem = sems[:2]
        ssem = sems[2:]
        wid = lax.axis_index("s") * info.num_cores + lax.axis_index("c")
        b0 = wid * BT

        pltpu.sync_copy(idx_hbm.at[:, pl.ds(b0, BT)], idx_v)

        def prep(si, buf):
            for k in range(BT // LANES):
                sl = pl.ds(k * LANES, LANES)
                ridx_v[buf, sl] = jax.lax.shift_right_logical(idx_v[si, sl], 1)

        def gather_desc(buf, make):
            return make(table_hbm.at[ridx_v.at[buf]], rows_v.at[buf], gsem[buf])

        def store_desc(si, buf, make):
            return make(out_v.at[buf], out_hbm.at[si, :, pl.ds(b0, BT)], ssem[buf])

        def transpose(si, buf):
            rows = rows_v.at[buf]
            for k in range(BT // LANES):
                sl = pl.ds(k * LANES, LANES)
                j_vec = jax.lax.iota(jnp.int32, LANES) + k * LANES
                c0 = jax.lax.shift_left(idx_v[si, sl] & 1, 6)
                for dd in range(d):
                    out_v[buf, dd, sl] = plsc.load_gather(rows, [j_vec, c0 + dd])

        prep(0, 0)
        gather_desc(0, pltpu.async_copy)

        def body(sp, carry):
            for buf in range(2):
                si = sp * 2 + buf
                nbuf = 1 - buf
                gather_desc(buf, pltpu.make_async_copy).wait()

                @pl.when(si + 1 < s)
                def _():
                    prep(si + 1, nbuf)
                    gather_desc(nbuf, pltpu.async_copy)

                @pl.when(si >= 2)
                def _():
                    store_desc(si - 2, buf, pltpu.make_async_copy).wait()

                transpose(si, buf)
                store_desc(si, buf, pltpu.async_copy)
            return carry

        lax.fori_loop(0, s // 2, body, 0)

        store_desc(s - 2, 0, pltpu.make_async_copy).wait()
        store_desc(s - 1, 1, pltpu.make_async_copy).wait()

    return gather_kernel


def kernel(src, src_attn_mask, embedding_table):
    b, s = src.shape
    v, d = embedding_table.shape
    table2 = embedding_table.reshape(v // 2, 2 * d)
    out = _make_gather(s, b, d)(table2, src.T)
    return out.transpose(2, 0, 1)


# speedup vs baseline: 1.4395x; 1.4395x over previous
import functools

import jax
import jax.numpy as jnp
from jax import lax
from jax.experimental import pallas as pl
from jax.experimental.pallas import tpu as pltpu
from jax.experimental.pallas import tpu_sc as plsc

LANES = 16
BT = 128


@functools.cache
def _make_gather(s: int, b: int, d: int):
    info = plsc.get_sparse_core_info()
    nw = info.num_cores * info.num_subcores
    assert b == BT * nw and d == 64 and s % 2 == 0

    mesh = plsc.VectorSubcoreMesh(core_axis_name="c", subcore_axis_name="s")

    @functools.partial(
        pl.kernel,
        mesh=mesh,
        out_type=jax.ShapeDtypeStruct((s, d, b), jnp.float32),
        scratch_types=[
            pltpu.VMEM((s, BT), jnp.int32),
            pltpu.VMEM((2, BT), jnp.int32),
            pltpu.VMEM((2, BT, 2 * d), jnp.float32),
            pltpu.VMEM((2, d, BT), jnp.float32),
        ]
        + [pltpu.SemaphoreType.DMA] * 4,
        compiler_params=pltpu.CompilerParams(needs_layout_passes=False),
    )
    def gather_kernel(table_hbm, idx_hbm, out_hbm, idx_v, ridx_v, rows_v, out_v, *sems):
        gsem = sems[:2]
        ssem = sems[2:]
        wid = lax.axis_index("s") * info.num_cores + lax.axis_index("c")
        b0 = wid * BT

        pltpu.sync_copy(idx_hbm.at[:, pl.ds(b0, BT)], idx_v)

        def prep(si, buf):
            for k in range(BT // LANES):
                sl = pl.ds(k * LANES, LANES)
                ridx_v[buf, sl] = jax.lax.shift_right_logical(idx_v[si, sl], 1)

        def gather_desc(buf, make):
            return make(table_hbm.at[ridx_v.at[buf]], rows_v.at[buf], gsem[buf])

        def store_desc(si, buf, make):
            return make(out_v.at[buf], out_hbm.at[si, :, pl.ds(b0, BT)], ssem[buf])

        def transpose(si, buf):
            rows = rows_v.at[buf]
            for k in range(BT // LANES):
                sl = pl.ds(k * LANES, LANES)
                j_vec = jax.lax.iota(jnp.int32, LANES) + k * LANES
                c0 = jax.lax.shift_left(idx_v[si, sl] & 1, 6)
                for d0 in range(0, d, 8):
                    vals = [
                        plsc.load_gather(rows, [j_vec, c0 + (d0 + i)])
                        for i in range(8)
                    ]
                    for i in range(8):
                        out_v[buf, d0 + i, sl] = vals[i]

        prep(0, 0)
        gather_desc(0, pltpu.async_copy)

        def body(sp, carry):
            for buf in range(2):
                si = sp * 2 + buf
                nbuf = 1 - buf
                gather_desc(buf, pltpu.make_async_copy).wait()

                @pl.when(si + 1 < s)
                def _():
                    prep(si + 1, nbuf)
                    gather_desc(nbuf, pltpu.async_copy)

                @pl.when(si >= 2)
                def _():
                    store_desc(si - 2, buf, pltpu.make_async_copy).wait()

                transpose(si, buf)
                store_desc(si, buf, pltpu.async_copy)
            return carry

        lax.fori_loop(0, s // 2, body, 0)

        store_desc(s - 2, 0, pltpu.make_async_copy).wait()
        store_desc(s - 1, 1, pltpu.make_async_copy).wait()

    return gather_kernel


def kernel(src, src_attn_mask, embedding_table):
    b, s = src.shape
    v, d = embedding_table.shape
    table2 = embedding_table.reshape(v // 2, 2 * d)
    out = _make_gather(s, b, d)(table2, src.T)
    return out.transpose(2, 0, 1)
